# Initial kernel scaffold; baseline (speedup 1.0000x reference)
import jax
import jax.numpy as jnp
from jax.experimental import pallas as pl


def kernel(inputs, targets):
    raise NotImplementedError("write your pallas kernel here")


# speedup vs baseline: 13.2231x; 13.2231x over previous
import jax
import jax.numpy as jnp
from jax.experimental import pallas as pl
from jax.experimental.pallas import tpu as pltpu

_ALPHA = 30.0
_K = 16
_N = 4096
_D = 512
_BLK = 256


def _knn_softmax_block(x_blk_ref, x_all_ref, t_col_ref, t_row_ref,
                       out_ref, sq_row_ref):
    i = pl.program_id(0)

    x_all = x_all_ref[...]

    @pl.when(i == 0)
    def _():
        ones = jnp.ones((1, _D), dtype=jnp.float32)
        sq_row_ref[...] = jax.lax.dot_general(
            ones, x_all * x_all, (((1,), (1,)), ((), ())),
            preferred_element_type=jnp.float32)

    x_blk = x_blk_ref[...]
    dot = jax.lax.dot_general(
        x_blk, x_all, (((1,), (1,)), ((), ())),
        precision=jax.lax.Precision.HIGHEST,
        preferred_element_type=jnp.float32)
    sq_blk = jnp.sum(x_blk * x_blk, axis=1, keepdims=True)
    d2 = jnp.maximum(sq_blk + sq_row_ref[...] - 2.0 * dot, 1e-12)

    row_ids = i * _BLK + jax.lax.broadcasted_iota(jnp.int32, (_BLK, _N), 0)
    col_ids = jax.lax.broadcasted_iota(jnp.int32, (_BLK, _N), 1)
    eye = row_ids == col_ids
    d2_off = jnp.where(eye, jnp.inf, d2)

    lb = jnp.full((_BLK, 1), -jnp.inf, dtype=jnp.float32)
    for _ in range(_K + 1):
        lb = jnp.min(jnp.where(d2_off > lb, d2_off, jnp.inf),
                     axis=1, keepdims=True)
    below = d2 < lb

    dist = jnp.sqrt(d2)
    expd = jnp.exp(_ALPHA * (1.0 - dist))

    same = t_col_ref[...] == t_row_ref[...]
    pos = same & (~eye)
    neg = ~same
    posb = pos & below
    negb = neg & below

    zero = jnp.zeros((), jnp.float32)
    pos_sum = jnp.sum(jnp.where(posb, expd, zero), axis=1, keepdims=True)
    neg_sum = jnp.sum(jnp.where(negb, expd, zero), axis=1, keepdims=True)
    npos_b = jnp.sum(jnp.where(posb, 1.0, zero), axis=1, keepdims=True)
    has_pn = npos_b > 0.0

    col_f = col_ids.astype(jnp.float32)
    fpos = jnp.min(jnp.where(pos, col_f, jnp.inf), axis=1, keepdims=True)
    fb = jnp.max(jnp.where(col_f == fpos, expd, -jnp.inf),
                 axis=1, keepdims=True)

    pos_logit = jnp.where(has_pn, pos_sum, fb)
    loss_i = -jnp.log(pos_logit / (pos_logit + neg_sum))

    any_pos = jnp.sum(jnp.where(pos, 1.0, zero), axis=1, keepdims=True) > 0.0
    any_neg = jnp.sum(jnp.where(neg, 1.0, zero), axis=1, keepdims=True) > 0.0
    valid = any_pos & any_neg

    lsum = jnp.sum(jnp.where(valid, loss_i, zero))
    vcnt = jnp.sum(jnp.where(valid, 1.0, zero))
    acnt = jnp.sum(jnp.where(valid & (loss_i < 0.6), 1.0, zero))

    lane = jax.lax.broadcasted_iota(jnp.int32, (1, 128), 1)
    vec = (jnp.where(lane == 0, lsum, zero)
           + jnp.where(lane == 1, vcnt, zero)
           + jnp.where(lane == 2, acnt, zero))

    @pl.when(i == 0)
    def _():
        out_ref[...] = jnp.zeros_like(out_ref)

    out_ref[...] += vec


def kernel(inputs, targets):
    n = inputs.shape[0]
    t_col = targets.reshape(n, 1)
    t_row = targets.reshape(1, n)
    out = pl.pallas_call(
        _knn_softmax_block,
        grid=(n // _BLK,),
        in_specs=[
            pl.BlockSpec((_BLK, _D), lambda i: (i, 0)),
            pl.BlockSpec((_N, _D), lambda i: (0, 0)),
            pl.BlockSpec((_BLK, 1), lambda i: (i, 0)),
            pl.BlockSpec((1, _N), lambda i: (0, 0)),
        ],
        out_specs=pl.BlockSpec((1, 128), lambda i: (0, 0)),
        out_shape=jax.ShapeDtypeStruct((1, 128), jnp.float32),
        scratch_shapes=[pltpu.VMEM((1, _N), jnp.float32)],
    )(inputs, inputs, t_col, t_row)
    loss = out[0, 0] / jnp.maximum(out[0, 1], 1.0)
    accuracy = out[0, 2] / jnp.float32(n)
    return loss, accuracy, jnp.float32(0.0), jnp.float32(0.0)


# speedup vs baseline: 14.5128x; 1.0975x over previous
import jax
import jax.numpy as jnp
from jax.experimental import pallas as pl
from jax.experimental.pallas import tpu as pltpu

_ALPHA = 30.0
_K = 16
_N = 4096
_D = 512
_BLK = 256
_C = 64


def _knn_softmax_block(x_blk_ref, x_all_ref, t_col_ref, t_row_ref,
                       out_ref, sq_row_ref, hist_ref):
    i = pl.program_id(0)

    x_all = x_all_ref[...]

    @pl.when(i == 0)
    def _():
        ones = jnp.ones((1, _D), dtype=jnp.float32)
        sq_row_ref[...] = jax.lax.dot_general(
            ones, x_all * x_all, (((1,), (1,)), ((), ())),
            preferred_element_type=jnp.float32)
        cls = jax.lax.broadcasted_iota(jnp.int32, (_C, 1), 0)
        onehot_all = (t_row_ref[...] == cls).astype(jnp.float32)
        hist_ref[...] = jnp.sum(onehot_all, axis=1, keepdims=True)

    x_blk = x_blk_ref[...]
    dot = jax.lax.dot_general(
        x_blk, x_all, (((1,), (1,)), ((), ())),
        precision=jax.lax.Precision.HIGHEST,
        preferred_element_type=jnp.float32)
    sq_blk = jnp.sum(x_blk * x_blk, axis=1, keepdims=True)
    d2 = jnp.maximum(sq_blk + sq_row_ref[...] - 2.0 * dot, 1e-12)

    row_ids = i * _BLK + jax.lax.broadcasted_iota(jnp.int32, (_BLK, _N), 0)
    col_ids = jax.lax.broadcasted_iota(jnp.int32, (_BLK, _N), 1)
    eye = row_ids == col_ids
    d2_off = jnp.where(eye, jnp.inf, d2)

    lb = jnp.full((_BLK, 1), -jnp.inf, dtype=jnp.float32)
    for _ in range(_K + 1):
        lb = jnp.min(jnp.where(d2_off > lb, d2_off, jnp.inf),
                     axis=1, keepdims=True)

    dist = jnp.sqrt(d2)
    expd = jnp.exp(_ALPHA - _ALPHA * dist)

    same = t_col_ref[...] == t_row_ref[...]
    pos = same & (~eye)
    obb = (d2 < lb) & (~eye)

    zero = jnp.zeros((), jnp.float32)
    eb = jnp.where(obb, expd, zero)
    pos_sum = jnp.sum(jnp.where(same, eb, zero), axis=1, keepdims=True)
    tot_sum = jnp.sum(eb, axis=1, keepdims=True)
    neg_sum = tot_sum - pos_sum
    has_pn = pos_sum > 0.0

    col_f = col_ids.astype(jnp.float32)
    fpos = jnp.min(jnp.where(pos, col_f, jnp.inf), axis=1, keepdims=True)
    fb = jnp.max(jnp.where(col_f == fpos, expd, -jnp.inf),
                 axis=1, keepdims=True)

    pos_logit = jnp.where(has_pn, pos_sum, fb)
    loss_i = -jnp.log(pos_logit / (pos_logit + neg_sum))

    cls_row = jax.lax.broadcasted_iota(jnp.int32, (1, _C), 1)
    onehot = (t_col_ref[...] == cls_row).astype(jnp.float32)
    cnt_same = jax.lax.dot_general(
        onehot, hist_ref[...], (((1,), (0,)), ((), ())),
        preferred_element_type=jnp.float32)
    valid = (cnt_same >= 2.0) & (cnt_same <= jnp.float32(_N - 1))

    lsum = jnp.sum(jnp.where(valid, loss_i, zero))
    vcnt = jnp.sum(jnp.where(valid, 1.0, zero))
    acnt = jnp.sum(jnp.where(valid & (loss_i < 0.6), 1.0, zero))

    lane = jax.lax.broadcasted_iota(jnp.int32, (1, 128), 1)
    vec = (jnp.where(lane == 0, lsum, zero)
           + jnp.where(lane == 1, vcnt, zero)
           + jnp.where(lane == 2, acnt, zero))

    @pl.when(i == 0)
    def _():
        out_ref[...] = jnp.zeros_like(out_ref)

    out_ref[...] += vec


def kernel(inputs, targets):
    n = inputs.shape[0]
    t_col = targets.reshape(n, 1)
    t_row = targets.reshape(1, n)
    out = pl.pallas_call(
        _knn_softmax_block,
        grid=(n // _BLK,),
        in_specs=[
            pl.BlockSpec((_BLK, _D), lambda i: (i, 0)),
            pl.BlockSpec((_N, _D), lambda i: (0, 0)),
            pl.BlockSpec((_BLK, 1), lambda i: (i, 0)),
            pl.BlockSpec((1, _N), lambda i: (0, 0)),
        ],
        out_specs=pl.BlockSpec((1, 128), lambda i: (0, 0)),
        out_shape=jax.ShapeDtypeStruct((1, 128), jnp.float32),
        scratch_shapes=[pltpu.VMEM((1, _N), jnp.float32),
                        pltpu.VMEM((_C, 1), jnp.float32)],
    )(inputs, inputs, t_col, t_row)
    loss = out[0, 0] / jnp.maximum(out[0, 1], 1.0)
    accuracy = out[0, 2] / jnp.float32(n)
    return loss, accuracy, jnp.float32(0.0), jnp.float32(0.0)


# speedup vs baseline: 21.3855x; 1.4736x over previous
import jax
import jax.numpy as jnp
from jax.experimental import pallas as pl
from jax.experimental.pallas import tpu as pltpu

_ALPHA = 30.0
_K = 16
_N = 4096
_D = 512
_BLK = 256
_C = 64
_S = 5


def _knn_softmax_block(x_blk_ref, x_all_ref, t_col_ref, t_row_ref,
                       out_ref, sq_row_ref, hist_ref):
    i = pl.program_id(0)

    x_all = x_all_ref[...]

    @pl.when(i == 0)
    def _():
        ones = jnp.ones((1, _D), dtype=jnp.float32)
        sq_row_ref[...] = jax.lax.dot_general(
            ones, x_all * x_all, (((1,), (1,)), ((), ())),
            preferred_element_type=jnp.float32)
        cls = jax.lax.broadcasted_iota(jnp.int32, (_C, 1), 0)
        onehot_all = (t_row_ref[...] == cls).astype(jnp.float32)
        hist_ref[...] = jnp.sum(onehot_all, axis=1, keepdims=True)

    x_blk = x_blk_ref[...]
    dot = jax.lax.dot_general(
        x_blk, x_all, (((1,), (1,)), ((), ())),
        preferred_element_type=jnp.float32)
    sq_blk = jnp.sum(x_blk * x_blk, axis=1, keepdims=True)
    d2 = jnp.maximum(sq_blk + sq_row_ref[...] - 2.0 * dot, 1e-12)

    row_ids = i * _BLK + jax.lax.broadcasted_iota(jnp.int32, (_BLK, _N), 0)
    col_ids = jax.lax.broadcasted_iota(jnp.int32, (_BLK, _N), 1)
    eye = row_ids == col_ids
    d2_off = jnp.where(eye, jnp.inf, d2)

    inf = jnp.float32(jnp.inf)
    parts = [d2_off[:, c * 128:(c + 1) * 128] for c in range(_N // 128)]
    m = parts[0]
    for p in parts[1:]:
        m = jnp.minimum(m, p)
    ms = [m]
    for _ in range(_S - 1):
        prev = ms[-1]
        m = jnp.full((_BLK, 128), jnp.inf, dtype=jnp.float32)
        for p in parts:
            m = jnp.minimum(m, jnp.where(p > prev, p, inf))
        ms.append(m)
    lb = jnp.full((_BLK, 1), -jnp.inf, dtype=jnp.float32)
    for _ in range(_K + 1):
        z = jnp.full((_BLK, 128), jnp.inf, dtype=jnp.float32)
        for y in ms:
            z = jnp.minimum(z, jnp.where(y > lb, y, inf))
        lb = jnp.min(z, axis=1, keepdims=True)

    zero = jnp.zeros((), jnp.float32)
    cnt = jnp.sum(jnp.where(d2_off < lb, 1.0, zero), axis=1, keepdims=True)

    def _full_select(_):
        lbf = jnp.full((_BLK, 1), -jnp.inf, dtype=jnp.float32)
        for _ in range(_K + 1):
            lbf = jnp.min(jnp.where(d2_off > lbf, d2_off, inf),
                          axis=1, keepdims=True)
        return lbf

    lb = jax.lax.cond(jnp.any(cnt > jnp.float32(_K)),
                      _full_select, lambda _: lb, 0)

    dist = jnp.sqrt(d2)
    expd = jnp.exp(_ALPHA - _ALPHA * dist)

    same = t_col_ref[...] == t_row_ref[...]
    pos = same & (~eye)
    obb = (d2 < lb) & (~eye)

    eb = jnp.where(obb, expd, zero)
    pos_sum = jnp.sum(jnp.where(same, eb, zero), axis=1, keepdims=True)
    tot_sum = jnp.sum(eb, axis=1, keepdims=True)
    neg_sum = tot_sum - pos_sum
    has_pn = pos_sum > 0.0

    col_f = col_ids.astype(jnp.float32)
    fpos = jnp.min(jnp.where(pos, col_f, jnp.inf), axis=1, keepdims=True)
    fb = jnp.max(jnp.where(col_f == fpos, expd, -jnp.inf),
                 axis=1, keepdims=True)

    pos_logit = jnp.where(has_pn, pos_sum, fb)
    loss_i = -jnp.log(pos_logit / (pos_logit + neg_sum))

    cls_row = jax.lax.broadcasted_iota(jnp.int32, (1, _C), 1)
    onehot = (t_col_ref[...] == cls_row).astype(jnp.float32)
    cnt_same = jax.lax.dot_general(
        onehot, hist_ref[...], (((1,), (0,)), ((), ())),
        preferred_element_type=jnp.float32)
    valid = (cnt_same >= 2.0) & (cnt_same <= jnp.float32(_N - 1))

    lsum = jnp.sum(jnp.where(valid, loss_i, zero))
    vcnt = jnp.sum(jnp.where(valid, 1.0, zero))
    acnt = jnp.sum(jnp.where(valid & (loss_i < 0.6), 1.0, zero))

    lane = jax.lax.broadcasted_iota(jnp.int32, (1, 128), 1)
    vec = (jnp.where(lane == 0, lsum, zero)
           + jnp.where(lane == 1, vcnt, zero)
           + jnp.where(lane == 2, acnt, zero))

    @pl.when(i == 0)
    def _():
        out_ref[...] = jnp.zeros_like(out_ref)

    out_ref[...] += vec


def kernel(inputs, targets):
    n = inputs.shape[0]
    t_col = targets.reshape(n, 1)
    t_row = targets.reshape(1, n)
    out = pl.pallas_call(
        _knn_softmax_block,
        grid=(n // _BLK,),
        in_specs=[
            pl.BlockSpec((_BLK, _D), lambda i: (i, 0)),
            pl.BlockSpec((_N, _D), lambda i: (0, 0)),
            pl.BlockSpec((_BLK, 1), lambda i: (i, 0)),
            pl.BlockSpec((1, _N), lambda i: (0, 0)),
        ],
        out_specs=pl.BlockSpec((1, 128), lambda i: (0, 0)),
        out_shape=jax.ShapeDtypeStruct((1, 128), jnp.float32),
        scratch_shapes=[pltpu.VMEM((1, _N), jnp.float32),
                        pltpu.VMEM((_C, 1), jnp.float32)],
    )(inputs, inputs, t_col, t_row)
    loss = out[0, 0] / jnp.maximum(out[0, 1], 1.0)
    accuracy = out[0, 2] / jnp.float32(n)
    return loss, accuracy, jnp.float32(0.0), jnp.float32(0.0)
